# Initial kernel scaffold; baseline (speedup 1.0000x reference)
import jax
import jax.numpy as jnp
from jax.experimental import pallas as pl


def kernel(relative_position_bias_table, relative_position_index):
    raise NotImplementedError("write your pallas kernel here")


# speedup vs baseline: 3.2660x; 3.2660x over previous
import functools

import jax
import jax.numpy as jnp
from jax import lax
from jax.experimental import pallas as pl
from jax.experimental.pallas import tpu as pltpu
from jax.experimental.pallas import tpu_sc as plsc

NUM_HEADS = 16
N = 256
B = N * N
NUM_WORKERS = 32
PER_WORKER = B // NUM_WORKERS
CHUNK = 128
NCHUNKS = PER_WORKER // CHUNK


def _sc_gather(table, idx2d):
    mesh = plsc.VectorSubcoreMesh(core_axis_name="c", subcore_axis_name="s")

    @functools.partial(
        pl.kernel,
        mesh=mesh,
        out_type=jax.ShapeDtypeStruct((B // CHUNK, CHUNK, NUM_HEADS), jnp.float32),
        scratch_types=[
            pltpu.VMEM((NCHUNKS, CHUNK), jnp.int32),
            pltpu.VMEM((NCHUNKS, CHUNK, NUM_HEADS), jnp.float32),
            pltpu.SemaphoreType.DMA,
        ],
        compiler_params=pltpu.CompilerParams(use_tc_tiling_on_sc=False),
    )
    def gather_kernel(table_hbm, idx_hbm, out_hbm, idx_v, rows_v, sem):
        wid = lax.axis_index("s") * 2 + lax.axis_index("c")
        row0 = wid * NCHUNKS
        pltpu.sync_copy(idx_hbm.at[pl.ds(row0, NCHUNKS)], idx_v)
        copies = [
            pltpu.async_copy(table_hbm.at[idx_v.at[j]], rows_v.at[j], sem)
            for j in range(NCHUNKS)
        ]
        for c in copies:
            c.wait()
        pltpu.sync_copy(rows_v, out_hbm.at[pl.ds(row0, NCHUNKS)])

    return gather_kernel(table, idx2d)


def kernel(relative_position_bias_table, relative_position_index):
    idx2d = relative_position_index.astype(jnp.int32).reshape(B // CHUNK, CHUNK)
    out = _sc_gather(relative_position_bias_table, idx2d)
    return out.reshape(N, N, NUM_HEADS)


# speedup vs baseline: 3.2682x; 1.0007x over previous
import functools

import jax
import jax.numpy as jnp
from jax import lax
from jax.experimental import pallas as pl
from jax.experimental.pallas import tpu as pltpu
from jax.experimental.pallas import tpu_sc as plsc

NUM_HEADS = 16
N = 256
NUM_WORKERS = 32
ROWS_PER_W = N // NUM_WORKERS
CHUNK = 128
CH_PER_ROW = N // CHUNK


def _sc_gather(table, idx):
    mesh = plsc.VectorSubcoreMesh(core_axis_name="c", subcore_axis_name="s")

    @functools.partial(
        pl.kernel,
        mesh=mesh,
        out_type=jax.ShapeDtypeStruct((N, N, NUM_HEADS), jnp.float32),
        scratch_types=[
            pltpu.VMEM((ROWS_PER_W, N), jnp.int32),
            pltpu.VMEM((ROWS_PER_W, N, NUM_HEADS), jnp.float32),
            pltpu.SemaphoreType.DMA,
        ],
        compiler_params=pltpu.CompilerParams(use_tc_tiling_on_sc=False),
    )
    def gather_kernel(table_hbm, idx_hbm, out_hbm, idx_v, rows_v, sem):
        wid = lax.axis_index("s") * 2 + lax.axis_index("c")
        row0 = wid * ROWS_PER_W
        pltpu.sync_copy(idx_hbm.at[pl.ds(row0, ROWS_PER_W)], idx_v)
        copies = [
            pltpu.async_copy(
                table_hbm.at[idx_v.at[r, pl.ds(cc * CHUNK, CHUNK)]],
                rows_v.at[r, pl.ds(cc * CHUNK, CHUNK)],
                sem,
            )
            for r in range(ROWS_PER_W)
            for cc in range(CH_PER_ROW)
        ]
        for c in copies:
            c.wait()
        pltpu.sync_copy(rows_v, out_hbm.at[pl.ds(row0, ROWS_PER_W)])

    return gather_kernel(table, idx)


def kernel(relative_position_bias_table, relative_position_index):
    idx = relative_position_index.astype(jnp.int32)
    return _sc_gather(relative_position_bias_table, idx)


# speedup vs baseline: 3.5178x; 1.0764x over previous
import functools

import jax
import jax.numpy as jnp
from jax import lax
from jax.experimental import pallas as pl
from jax.experimental.pallas import tpu as pltpu
from jax.experimental.pallas import tpu_sc as plsc

NUM_HEADS = 16
N = 256
NUM_WORKERS = 32
ROWS_PER_W = N // NUM_WORKERS
CHUNK = 128
BLK = N * NUM_HEADS


def _sc_gather(table, idx):
    mesh = plsc.VectorSubcoreMesh(core_axis_name="c", subcore_axis_name="s")

    @functools.partial(
        pl.kernel,
        mesh=mesh,
        out_type=jax.ShapeDtypeStruct((N, BLK), jnp.float32),
        scratch_types=[
            pltpu.VMEM((ROWS_PER_W, N), jnp.int32),
            pltpu.VMEM((N, NUM_HEADS), jnp.float32),
            pltpu.VMEM((BLK,), jnp.float32),
            pltpu.SemaphoreType.DMA,
        ],
        compiler_params=pltpu.CompilerParams(
            use_tc_tiling_on_sc=False, needs_layout_passes=False),
    )
    def gather_kernel(table_hbm, idx_hbm, out_hbm, idx_v, stage_v, blk_v, sem):
        wid = lax.axis_index("s") * 2 + lax.axis_index("c")
        row0 = wid * ROWS_PER_W
        pltpu.sync_copy(idx_hbm.at[pl.ds(row0, ROWS_PER_W)], idx_v)

        lanes = jax.lax.iota(jnp.int32, 16)
        addrmap = (lanes >> 3) * 2048 + (lanes & 7) * 128

        def body(a, _):
            c0 = pltpu.async_copy(
                table_hbm.at[idx_v.at[a, pl.ds(0, CHUNK)]],
                stage_v.at[pl.ds(0, CHUNK)], sem)
            c1 = pltpu.async_copy(
                table_hbm.at[idx_v.at[a, pl.ds(CHUNK, CHUNK)]],
                stage_v.at[pl.ds(CHUNK, CHUNK)], sem)
            c0.wait()
            c1.wait()
            for b in range(N):
                vals = stage_v[b, :]
                plsc.store_scatter(
                    blk_v, [addrmap + ((b // CHUNK) * 1024 + (b % CHUNK))], vals)
            out_row = row0 + a
            pltpu.sync_copy(blk_v, out_hbm.at[out_row])
            return ()

        lax.fori_loop(0, ROWS_PER_W, body, (), unroll=False)

    return gather_kernel(table, idx)


def kernel(relative_position_bias_table, relative_position_index):
    idx = relative_position_index.astype(jnp.int32)
    out = _sc_gather(relative_position_bias_table, idx)
    return (out.reshape(N, 2, 2, 8, 128)
               .transpose(0, 2, 4, 1, 3)
               .reshape(N, N, NUM_HEADS))


# speedup vs baseline: 5.7719x; 1.6408x over previous
import functools

import jax
import jax.numpy as jnp
from jax import lax
from jax.experimental import pallas as pl
from jax.experimental.pallas import tpu as pltpu
from jax.experimental.pallas import tpu_sc as plsc

NUM_HEADS = 16
N = 256
TABLE_WORDS = 961 * NUM_HEADS
NUM_WORKERS = 32
ROWS_PER_W = N // NUM_WORKERS
IDX_PER_W = ROWS_PER_W * N
BLK = N * NUM_HEADS
GROUPS = IDX_PER_W // 16


def _sc_gather(table_flat, idx_flat):
    mesh = plsc.VectorSubcoreMesh(core_axis_name="c", subcore_axis_name="s")

    @functools.partial(
        pl.kernel,
        mesh=mesh,
        out_type=jax.ShapeDtypeStruct((N * BLK,), jnp.float32),
        scratch_types=[
            pltpu.VMEM((TABLE_WORDS,), jnp.float32),
            pltpu.VMEM((IDX_PER_W,), jnp.int32),
            pltpu.VMEM((ROWS_PER_W * BLK,), jnp.float32),
            pltpu.SemaphoreType.DMA,
        ],
        compiler_params=pltpu.CompilerParams(
            use_tc_tiling_on_sc=False, needs_layout_passes=False),
    )
    def gather_kernel(table_hbm, idx_hbm, out_hbm, table_v, idx_v, blk_v, sem):
        wid = lax.axis_index("s") * 2 + lax.axis_index("c")
        ct = pltpu.async_copy(table_hbm, table_v, sem)
        ci = pltpu.async_copy(idx_hbm.at[pl.ds(wid * IDX_PER_W, IDX_PER_W)],
                              idx_v, sem)
        ct.wait()
        ci.wait()

        def body(r, _):
            idx16 = idx_v[pl.ds(r * 16, 16)]
            flat = idx16 * NUM_HEADS
            base = (r >> 4) * BLK + ((r >> 3) & 1) * 1024 + (r & 7) * 16
            for c in range(NUM_HEADS):
                vals = plsc.load_gather(table_v, [flat + c])
                blk_v[pl.ds(base + (c >> 3) * 2048 + (c & 7) * 128, 16)] = vals
            return ()

        lax.fori_loop(0, GROUPS, body, (), unroll=False)
        pltpu.sync_copy(
            blk_v, out_hbm.at[pl.ds(wid * ROWS_PER_W * BLK, ROWS_PER_W * BLK)])

    return gather_kernel(table_flat, idx_flat)


def kernel(relative_position_bias_table, relative_position_index):
    table_flat = relative_position_bias_table.reshape(-1)
    idx_flat = relative_position_index.astype(jnp.int32).reshape(-1)
    out = _sc_gather(table_flat, idx_flat)
    return (out.reshape(N, 2, 2, 8, 128)
               .transpose(0, 2, 4, 1, 3)
               .reshape(N, N, NUM_HEADS))
